# Initial kernel scaffold; baseline (speedup 1.0000x reference)
import jax
import jax.numpy as jnp
from jax.experimental import pallas as pl


def kernel(x, edge_index, batch, y, W1, b1, g1, be1, W2, b2, g2, be2, LW1, Lb1, g3, be3, LW2, Lb2):
    raise NotImplementedError("write your pallas kernel here")


# speedup vs baseline: 15.0459x; 15.0459x over previous
import functools

import jax
import jax.numpy as jnp
from jax import lax
from jax.experimental import pallas as pl
from jax.experimental.pallas import tpu as pltpu
from jax.experimental.pallas import tpu_sc as plsc

N = 10000
E = 320000
F_IN = 128
D1 = 256
D2 = 256
D3 = 128
C = 10
G = 64

NC = 2
NS = 16
NW = NC * NS

DR = 80
DCOL = 128
EW_DEG = E // NW
DEG_CH = EW_DEG // 16

ES = E // NS
CHUNK = 80
NCH = ES // CHUNK
NBLK = 50
NB = NCH // NBLK
RPS = N // NS

HALF = 128

def _deg_body(dst_hbm, rowid_hbm, out_hbm, dstv, partial, rowv, shared):
    c = lax.axis_index("c")
    s = lax.axis_index("s")
    w = c * NS + s
    zeros16 = jnp.zeros((16,), jnp.float32)

    def zrow(i, carry):
        for j in range(DCOL // 16):
            partial[i, pl.ds(j * 16, 16)] = zeros16
        return carry

    lax.fori_loop(0, DR, zrow, 0)
    rps = DR // NS
    pltpu.sync_copy(partial.at[pl.ds(0, rps)], shared.at[pl.ds(s * rps, rps)])
    pltpu.sync_copy(rowid_hbm, rowv)
    pltpu.sync_copy(dst_hbm.at[w], dstv)

    ones16 = jnp.ones((16,), jnp.float32)

    def body(i, carry):
        idx = dstv[i, :]
        r = lax.shift_right_logical(idx, 7)
        col = lax.bitwise_and(idx, 127)
        plsc.addupdate_scatter(partial, [r, col], ones16)
        return carry

    lax.fori_loop(0, DEG_CH, body, 0)
    plsc.subcore_barrier()
    pltpu.sync_copy(partial, shared.at[rowv], add=True)
    plsc.subcore_barrier()
    pltpu.sync_copy(shared.at[pl.ds(s * rps, rps)],
                    out_hbm.at[pl.ds(c * DR + s * rps, rps)])


def _scatter_body(hp_hbm, src_hbm, dst_hbm, out_hbm,
                  srcv, dstv, buf0, buf1, sem0, sem1, acc):
    c = lax.axis_index("c")
    s = lax.axis_index("s")
    w = c * NS + s
    pltpu.sync_copy(hp_hbm.at[pl.ds(c * N + s * RPS, RPS)],
                    acc.at[pl.ds(s * RPS, RPS)])
    plsc.subcore_barrier()

    def outer(b, carry):
        pltpu.sync_copy(src_hbm.at[w, pl.ds(b * NBLK, NBLK)], srcv)
        pltpu.sync_copy(dst_hbm.at[s, pl.ds(b * NBLK, NBLK)], dstv)

        def body(i, carry2):
            j0 = 2 * i
            j1 = j0 + 1
            d0 = pltpu.async_copy(hp_hbm.at[srcv.at[j0]], buf0, sem0)
            d1 = pltpu.async_copy(hp_hbm.at[srcv.at[j1]], buf1, sem1)
            d0.wait()
            pltpu.sync_copy(buf0, acc.at[dstv.at[j0]], add=True)
            d1.wait()
            pltpu.sync_copy(buf1, acc.at[dstv.at[j1]], add=True)
            return carry2

        lax.fori_loop(0, NBLK // 2, body, 0)
        return carry

    lax.fori_loop(0, NB, outer, 0)
    plsc.subcore_barrier()
    pltpu.sync_copy(acc.at[pl.ds(s * RPS, RPS)],
                    out_hbm.at[pl.ds(c * N + s * RPS, RPS)])


@functools.lru_cache(maxsize=None)
def _sc_kernels():
    mesh = plsc.VectorSubcoreMesh(
        core_axis_name="c", subcore_axis_name="s",
        num_cores=NC, num_subcores=NS)
    params = pltpu.CompilerParams(
        use_tc_tiling_on_sc=False, needs_layout_passes=False)
    deg = pl.kernel(
        _deg_body,
        out_type=jax.ShapeDtypeStruct((NC * DR, DCOL), jnp.float32),
        mesh=mesh,
        scratch_types=[
            pltpu.VMEM((DEG_CH, 16), jnp.int32),
            pltpu.VMEM((DR, DCOL), jnp.float32),
            pltpu.VMEM((DR,), jnp.int32),
            pltpu.VMEM_SHARED((DR, DCOL), jnp.float32),
        ],
        compiler_params=params,
    )
    scat = pl.kernel(
        _scatter_body,
        out_type=jax.ShapeDtypeStruct((NC * N, HALF), jnp.float32),
        mesh=mesh,
        scratch_types=[
            pltpu.VMEM((NBLK, CHUNK), jnp.int32),
            pltpu.VMEM((NBLK, CHUNK), jnp.int32),
            pltpu.VMEM((CHUNK, HALF), jnp.float32),
            pltpu.VMEM((CHUNK, HALF), jnp.float32),
            pltpu.SemaphoreType.DMA,
            pltpu.SemaphoreType.DMA,
            pltpu.VMEM_SHARED((N, HALF), jnp.float32),
        ],
        compiler_params=params,
    )
    return deg, scat


def _dinv_col(dp):
    return lax.rsqrt(dp[:, 0:1] + dp[:, 1:2] + 1.0)


def _prep_body(x_ref, w1_ref, dp_ref, hp_ref):
    dinv = _dinv_col(dp_ref[...])
    h = jnp.dot(x_ref[...], w1_ref[...], preferred_element_type=jnp.float32)
    hp = h * dinv
    hp_ref[0] = hp[:, :HALF]
    hp_ref[1] = hp[:, HALF:]


def _bn(a, gamma, beta):
    mu = jnp.mean(a, axis=0, keepdims=True)
    var = jnp.mean((a - mu) * (a - mu), axis=0, keepdims=True)
    return (a - mu) * lax.rsqrt(var + 1e-5) * gamma + beta


def _mid_body(s_ref, dp_ref, b1_ref, g1_ref, be1_ref, w2_ref, out_ref):
    dinv = _dinv_col(dp_ref[...])
    sc = jnp.concatenate([s_ref[0], s_ref[1]], axis=1)
    a = sc * dinv + b1_ref[...]
    hr = jnp.maximum(_bn(a, g1_ref[...], be1_ref[...]), 0.0)
    h2 = jnp.dot(hr, w2_ref[...], preferred_element_type=jnp.float32)
    hp2 = h2 * dinv
    out_ref[0] = hp2[:, :HALF]
    out_ref[1] = hp2[:, HALF:]


def _final_body(s_ref, dp_ref, b2_ref, g2_ref, be2_ref, batch_ref,
                lw1_ref, lb1_ref, g3_ref, be3_ref, lw2_ref, lb2_ref,
                out_ref, pooled_ref):
    dinv = _dinv_col(dp_ref[...])
    sc = jnp.concatenate([s_ref[0], s_ref[1]], axis=1)
    a = sc * dinv + b2_ref[...]
    h = jnp.maximum(_bn(a, g2_ref[...], be2_ref[...]), 0.0)

    bids = batch_ref[...]

    def pool_body(g, carry):
        mask = bids == g
        m = jnp.max(jnp.where(mask, h, -jnp.inf), axis=0, keepdims=True)
        pooled_ref[pl.ds(g, 1), :] = m
        return carry

    lax.fori_loop(0, G, pool_body, 0)
    pooled = pooled_ref[...]

    p1 = jnp.dot(pooled, lw1_ref[...],
                 preferred_element_type=jnp.float32) + lb1_ref[...]
    p2 = jnp.maximum(_bn(p1, g3_ref[...], be3_ref[...]), 0.0)
    logits = jnp.dot(p2, lw2_ref[...],
                     preferred_element_type=jnp.float32) + lb2_ref[...]
    mx = jnp.max(logits, axis=1, keepdims=True)
    lse = jnp.log(jnp.sum(jnp.exp(logits - mx), axis=1, keepdims=True)) + mx
    out_ref[...] = logits - lse


def kernel(x, edge_index, batch, y, W1, b1, g1, be1, W2, b2, g2, be2,
           LW1, Lb1, g3, be3, LW2, Lb2):
    src = edge_index[0]
    dst = edge_index[1]
    deg_kernel, scatter_kernel = _sc_kernels()

    dst_deg = dst.reshape(NW, DEG_CH, 16)
    rowids = jnp.arange(DR, dtype=jnp.int32)
    degs = deg_kernel(dst_deg, rowids)
    deg_pair = degs.reshape(NC, DR * DCOL)[:, :N].T

    hp1 = _prep_call(x, W1, deg_pair)

    src_r = src.reshape(NS, NCH, CHUNK)
    src32 = jnp.concatenate([src_r, src_r + N], axis=0)
    dst16 = dst.reshape(NS, NCH, CHUNK)

    s1 = scatter_kernel(hp1.reshape(NC * N, HALF), src32, dst16)

    hp2 = _mid_call(s1.reshape(NC, N, HALF), deg_pair,
                    b1[None], g1[None], be1[None], W2)
    s2 = scatter_kernel(hp2.reshape(NC * N, HALF), src32, dst16)

    out = _final_call(s2.reshape(NC, N, HALF), deg_pair,
                      b2[None], g2[None], be2[None], batch[:, None],
                      LW1, Lb1[None], g3[None], be3[None], LW2, Lb2[None])
    return (out, y.astype(jnp.int32))


_prep_call = pl.pallas_call(
    _prep_body,
    out_shape=jax.ShapeDtypeStruct((NC, N, HALF), jnp.float32),
)

_mid_call = pl.pallas_call(
    _mid_body,
    out_shape=jax.ShapeDtypeStruct((NC, N, HALF), jnp.float32),
)

_final_call = pl.pallas_call(
    _final_body,
    out_shape=jax.ShapeDtypeStruct((G, C), jnp.float32),
    scratch_shapes=[pltpu.VMEM((G, D2), jnp.float32)],
)


# speedup vs baseline: 18.7926x; 1.2490x over previous
import functools

import jax
import jax.numpy as jnp
from jax import lax
from jax.experimental import pallas as pl
from jax.experimental.pallas import tpu as pltpu
from jax.experimental.pallas import tpu_sc as plsc

N = 10000
E = 320000
F_IN = 128
D1 = 256
D2 = 256
D3 = 128
C = 10
G = 64

NC = 2
NS = 16
NW = NC * NS

DR = 80
DCOL = 128
EW_DEG = E // NW
DEG_CH = EW_DEG // 16

ES = E // NS
CHUNK = 80
NCH = ES // CHUNK
NBLK = 50
NB = NCH // NBLK
RPS = N // NS

HALF = 128

def _deg_body(dst_hbm, rowid_hbm, out_hbm, dstv, partial, rowv, shared):
    c = lax.axis_index("c")
    s = lax.axis_index("s")
    w = c * NS + s
    zeros16 = jnp.zeros((16,), jnp.float32)

    def zrow(i, carry):
        for j in range(DCOL // 16):
            partial[i, pl.ds(j * 16, 16)] = zeros16
        return carry

    lax.fori_loop(0, DR, zrow, 0)
    rps = DR // NS
    pltpu.sync_copy(partial.at[pl.ds(0, rps)], shared.at[pl.ds(s * rps, rps)])
    pltpu.sync_copy(rowid_hbm, rowv)
    pltpu.sync_copy(dst_hbm.at[w], dstv)

    ones16 = jnp.ones((16,), jnp.float32)

    def body(i, carry):
        idx = dstv[i, :]
        r = lax.shift_right_logical(idx, 7)
        col = lax.bitwise_and(idx, 127)
        plsc.addupdate_scatter(partial, [r, col], ones16)
        return carry

    lax.fori_loop(0, DEG_CH, body, 0)
    plsc.subcore_barrier()
    pltpu.sync_copy(partial, shared.at[rowv], add=True)
    plsc.subcore_barrier()
    pltpu.sync_copy(shared.at[pl.ds(s * rps, rps)],
                    out_hbm.at[pl.ds(c * DR + s * rps, rps)])


def _scatter_body(hp_hbm, src_hbm, dst_hbm, out_hbm,
                  srcv, dstv, buf0, buf1, buf2, buf3,
                  sem0, sem1, sem2, sem3, acc):
    c = lax.axis_index("c")
    s = lax.axis_index("s")
    w = c * NS + s
    pltpu.sync_copy(hp_hbm.at[pl.ds(c * N + s * RPS, RPS)],
                    acc.at[pl.ds(s * RPS, RPS)])
    plsc.subcore_barrier()

    bufs = (buf0, buf1, buf2, buf3)
    sems = (sem0, sem1, sem2, sem3)

    def gather(j, q):
        return pltpu.async_copy(hp_hbm.at[srcv.at[j]], bufs[q], sems[q])

    def drain_add(j, q):
        pltpu.make_async_copy(hp_hbm.at[srcv.at[j]], bufs[q], sems[q]).wait()
        pltpu.sync_copy(bufs[q], acc.at[dstv.at[j]], add=True)

    def outer(b, carry):
        pltpu.sync_copy(src_hbm.at[w, pl.ds(b * NBLK, NBLK)], srcv)
        pltpu.sync_copy(dst_hbm.at[s, pl.ds(b * NBLK, NBLK)], dstv)
        gather(0, 0)
        gather(1, 1)

        def body(k, carry2):
            j = 4 * k
            drain_add(j, 0)
            gather(j + 2, 2)
            drain_add(j + 1, 1)
            gather(j + 3, 3)
            drain_add(j + 2, 2)
            gather(j + 4, 0)
            drain_add(j + 3, 3)
            gather(j + 5, 1)
            return carry2

        lax.fori_loop(0, NBLK // 4 - 1, body, 0)
        j = NBLK - 6
        drain_add(j, 0)
        gather(j + 2, 2)
        drain_add(j + 1, 1)
        gather(j + 3, 3)
        drain_add(j + 2, 2)
        gather(j + 4, 0)
        drain_add(j + 3, 3)
        gather(j + 5, 1)
        drain_add(j + 4, 0)
        drain_add(j + 5, 1)
        return carry

    lax.fori_loop(0, NB, outer, 0)
    plsc.subcore_barrier()
    pltpu.sync_copy(acc.at[pl.ds(s * RPS, RPS)],
                    out_hbm.at[pl.ds(c * N + s * RPS, RPS)])


@functools.lru_cache(maxsize=None)
def _sc_kernels():
    mesh = plsc.VectorSubcoreMesh(
        core_axis_name="c", subcore_axis_name="s",
        num_cores=NC, num_subcores=NS)
    params = pltpu.CompilerParams(
        use_tc_tiling_on_sc=False, needs_layout_passes=False)
    deg = pl.kernel(
        _deg_body,
        out_type=jax.ShapeDtypeStruct((NC * DR, DCOL), jnp.float32),
        mesh=mesh,
        scratch_types=[
            pltpu.VMEM((DEG_CH, 16), jnp.int32),
            pltpu.VMEM((DR, DCOL), jnp.float32),
            pltpu.VMEM((DR,), jnp.int32),
            pltpu.VMEM_SHARED((DR, DCOL), jnp.float32),
        ],
        compiler_params=params,
    )
    scat = pl.kernel(
        _scatter_body,
        out_type=jax.ShapeDtypeStruct((NC * N, HALF), jnp.float32),
        mesh=mesh,
        scratch_types=[
            pltpu.VMEM((NBLK, CHUNK), jnp.int32),
            pltpu.VMEM((NBLK, CHUNK), jnp.int32),
            pltpu.VMEM((CHUNK, HALF), jnp.float32),
            pltpu.VMEM((CHUNK, HALF), jnp.float32),
            pltpu.VMEM((CHUNK, HALF), jnp.float32),
            pltpu.VMEM((CHUNK, HALF), jnp.float32),
            pltpu.SemaphoreType.DMA,
            pltpu.SemaphoreType.DMA,
            pltpu.SemaphoreType.DMA,
            pltpu.SemaphoreType.DMA,
            pltpu.VMEM_SHARED((N, HALF), jnp.float32),
        ],
        compiler_params=params,
    )
    return deg, scat


def _dinv_col(dp):
    return lax.rsqrt(dp[:, 0:1] + dp[:, 1:2] + 1.0)


def _prep_body(x_ref, w1_ref, dp_ref, hp_ref):
    dinv = _dinv_col(dp_ref[...])
    h = jnp.dot(x_ref[...], w1_ref[...], preferred_element_type=jnp.float32)
    hp = h * dinv
    hp_ref[0] = hp[:, :HALF]
    hp_ref[1] = hp[:, HALF:]


def _bn(a, gamma, beta):
    mu = jnp.mean(a, axis=0, keepdims=True)
    var = jnp.mean((a - mu) * (a - mu), axis=0, keepdims=True)
    return (a - mu) * lax.rsqrt(var + 1e-5) * gamma + beta


def _mid_body(s_ref, dp_ref, b1_ref, g1_ref, be1_ref, w2_ref, out_ref):
    dinv = _dinv_col(dp_ref[...])
    sc = jnp.concatenate([s_ref[0], s_ref[1]], axis=1)
    a = sc * dinv + b1_ref[...]
    hr = jnp.maximum(_bn(a, g1_ref[...], be1_ref[...]), 0.0)
    h2 = jnp.dot(hr, w2_ref[...], preferred_element_type=jnp.float32)
    hp2 = h2 * dinv
    out_ref[0] = hp2[:, :HALF]
    out_ref[1] = hp2[:, HALF:]


def _final_body(s_ref, dp_ref, b2_ref, g2_ref, be2_ref, batch_ref,
                lw1_ref, lb1_ref, g3_ref, be3_ref, lw2_ref, lb2_ref,
                out_ref, pooled_ref):
    dinv = _dinv_col(dp_ref[...])
    sc = jnp.concatenate([s_ref[0], s_ref[1]], axis=1)
    a = sc * dinv + b2_ref[...]
    h = jnp.maximum(_bn(a, g2_ref[...], be2_ref[...]), 0.0)

    bids = batch_ref[...]

    def pool_body(g, carry):
        mask = bids == g
        m = jnp.max(jnp.where(mask, h, -jnp.inf), axis=0, keepdims=True)
        pooled_ref[pl.ds(g, 1), :] = m
        return carry

    lax.fori_loop(0, G, pool_body, 0)
    pooled = pooled_ref[...]

    p1 = jnp.dot(pooled, lw1_ref[...],
                 preferred_element_type=jnp.float32) + lb1_ref[...]
    p2 = jnp.maximum(_bn(p1, g3_ref[...], be3_ref[...]), 0.0)
    logits = jnp.dot(p2, lw2_ref[...],
                     preferred_element_type=jnp.float32) + lb2_ref[...]
    mx = jnp.max(logits, axis=1, keepdims=True)
    lse = jnp.log(jnp.sum(jnp.exp(logits - mx), axis=1, keepdims=True)) + mx
    out_ref[...] = logits - lse


def kernel(x, edge_index, batch, y, W1, b1, g1, be1, W2, b2, g2, be2,
           LW1, Lb1, g3, be3, LW2, Lb2):
    src = edge_index[0]
    dst = edge_index[1]
    deg_kernel, scatter_kernel = _sc_kernels()

    dst_deg = dst.reshape(NW, DEG_CH, 16)
    rowids = jnp.arange(DR, dtype=jnp.int32)
    degs = deg_kernel(dst_deg, rowids)
    deg_pair = degs.reshape(NC, DR * DCOL)[:, :N].T

    hp1 = _prep_call(x, W1, deg_pair)

    src_r = src.reshape(NS, NCH, CHUNK)
    src32 = jnp.concatenate([src_r, src_r + N], axis=0)
    dst16 = dst.reshape(NS, NCH, CHUNK)

    s1 = scatter_kernel(hp1.reshape(NC * N, HALF), src32, dst16)

    hp2 = _mid_call(s1.reshape(NC, N, HALF), deg_pair,
                    b1[None], g1[None], be1[None], W2)
    s2 = scatter_kernel(hp2.reshape(NC * N, HALF), src32, dst16)

    out = _final_call(s2.reshape(NC, N, HALF), deg_pair,
                      b2[None], g2[None], be2[None], batch[:, None],
                      LW1, Lb1[None], g3[None], be3[None], LW2, Lb2[None])
    return (out, y.astype(jnp.int32))


_prep_call = pl.pallas_call(
    _prep_body,
    out_shape=jax.ShapeDtypeStruct((NC, N, HALF), jnp.float32),
)

_mid_call = pl.pallas_call(
    _mid_body,
    out_shape=jax.ShapeDtypeStruct((NC, N, HALF), jnp.float32),
)

_final_call = pl.pallas_call(
    _final_body,
    out_shape=jax.ShapeDtypeStruct((G, C), jnp.float32),
    scratch_shapes=[pltpu.VMEM((G, D2), jnp.float32)],
)


# speedup vs baseline: 21.7739x; 1.1586x over previous
import functools

import jax
import jax.numpy as jnp
from jax import lax
from jax.experimental import pallas as pl
from jax.experimental.pallas import tpu as pltpu
from jax.experimental.pallas import tpu_sc as plsc

N = 10000
E = 320000
F_IN = 128
D1 = 256
D2 = 256
D3 = 128
C = 10
G = 64

NC = 2
NS = 16
NW = NC * NS

DR = 80
DCOL = 128
EW_DEG = E // NW
DEG_CH = EW_DEG // 16

ES = E // NS
CHUNK = 80
NCH = ES // CHUNK
NBLK = 50
NB = NCH // NBLK
RPS = N // NS

HALF = 128

PBLK = 80
NPB = N // PBLK

def _deg_body(dst_hbm, rowid_hbm, out_hbm, dstv, partial, rowv, shared):
    c = lax.axis_index("c")
    s = lax.axis_index("s")
    w = c * NS + s
    zeros16 = jnp.zeros((16,), jnp.float32)

    def zrow(i, carry):
        for j in range(DCOL // 16):
            partial[i, pl.ds(j * 16, 16)] = zeros16
        return carry

    lax.fori_loop(0, DR, zrow, 0)
    rps = DR // NS
    pltpu.sync_copy(partial.at[pl.ds(0, rps)], shared.at[pl.ds(s * rps, rps)])
    pltpu.sync_copy(rowid_hbm, rowv)
    pltpu.sync_copy(dst_hbm.at[w], dstv)

    ones16 = jnp.ones((16,), jnp.float32)

    def body(i, carry):
        idx = dstv[i, :]
        r = lax.shift_right_logical(idx, 7)
        col = lax.bitwise_and(idx, 127)
        plsc.addupdate_scatter(partial, [r, col], ones16)
        return carry

    lax.fori_loop(0, DEG_CH, body, 0)
    plsc.subcore_barrier()
    pltpu.sync_copy(partial, shared.at[rowv], add=True)
    plsc.subcore_barrier()
    pltpu.sync_copy(shared.at[pl.ds(s * rps, rps)],
                    out_hbm.at[pl.ds(c * DR + s * rps, rps)])


def _scatter_body(hp_hbm, src_hbm, dst_hbm, out_hbm,
                  srcv, dstv, buf0, buf1, buf2, buf3,
                  sem0, sem1, sem2, sem3, acc):
    c = lax.axis_index("c")
    s = lax.axis_index("s")
    w = c * NS + s
    pltpu.sync_copy(hp_hbm.at[pl.ds(c * N + s * RPS, RPS)],
                    acc.at[pl.ds(s * RPS, RPS)])
    plsc.subcore_barrier()

    bufs = (buf0, buf1, buf2, buf3)
    sems = (sem0, sem1, sem2, sem3)

    def gather(j, q):
        return pltpu.async_copy(hp_hbm.at[srcv.at[j]], bufs[q], sems[q])

    def drain_add(j, q):
        pltpu.make_async_copy(hp_hbm.at[srcv.at[j]], bufs[q], sems[q]).wait()
        pltpu.sync_copy(bufs[q], acc.at[dstv.at[j]], add=True)

    def outer(b, carry):
        pltpu.sync_copy(src_hbm.at[w, pl.ds(b * NBLK, NBLK)], srcv)
        pltpu.sync_copy(dst_hbm.at[s, pl.ds(b * NBLK, NBLK)], dstv)
        gather(0, 0)
        gather(1, 1)

        def body(k, carry2):
            j = 4 * k
            drain_add(j, 0)
            gather(j + 2, 2)
            drain_add(j + 1, 1)
            gather(j + 3, 3)
            drain_add(j + 2, 2)
            gather(j + 4, 0)
            drain_add(j + 3, 3)
            gather(j + 5, 1)
            return carry2

        lax.fori_loop(0, NBLK // 4 - 1, body, 0)
        j = NBLK - 6
        drain_add(j, 0)
        gather(j + 2, 2)
        drain_add(j + 1, 1)
        gather(j + 3, 3)
        drain_add(j + 2, 2)
        gather(j + 4, 0)
        drain_add(j + 3, 3)
        gather(j + 5, 1)
        drain_add(j + 4, 0)
        drain_add(j + 5, 1)
        return carry

    lax.fori_loop(0, NB, outer, 0)
    plsc.subcore_barrier()
    pltpu.sync_copy(acc.at[pl.ds(s * RPS, RPS)],
                    out_hbm.at[pl.ds(c * N + s * RPS, RPS)])


@functools.lru_cache(maxsize=None)
def _sc_kernels():
    mesh = plsc.VectorSubcoreMesh(
        core_axis_name="c", subcore_axis_name="s",
        num_cores=NC, num_subcores=NS)
    params = pltpu.CompilerParams(
        use_tc_tiling_on_sc=False, needs_layout_passes=False)
    deg = pl.kernel(
        _deg_body,
        out_type=jax.ShapeDtypeStruct((NC * DR, DCOL), jnp.float32),
        mesh=mesh,
        scratch_types=[
            pltpu.VMEM((DEG_CH, 16), jnp.int32),
            pltpu.VMEM((DR, DCOL), jnp.float32),
            pltpu.VMEM((DR,), jnp.int32),
            pltpu.VMEM_SHARED((DR, DCOL), jnp.float32),
        ],
        compiler_params=params,
    )
    scat = pl.kernel(
        _scatter_body,
        out_type=jax.ShapeDtypeStruct((NC * N, HALF), jnp.float32),
        mesh=mesh,
        scratch_types=[
            pltpu.VMEM((NBLK, CHUNK), jnp.int32),
            pltpu.VMEM((NBLK, CHUNK), jnp.int32),
            pltpu.VMEM((CHUNK, HALF), jnp.float32),
            pltpu.VMEM((CHUNK, HALF), jnp.float32),
            pltpu.VMEM((CHUNK, HALF), jnp.float32),
            pltpu.VMEM((CHUNK, HALF), jnp.float32),
            pltpu.SemaphoreType.DMA,
            pltpu.SemaphoreType.DMA,
            pltpu.SemaphoreType.DMA,
            pltpu.SemaphoreType.DMA,
            pltpu.VMEM_SHARED((N, HALF), jnp.float32),
        ],
        compiler_params=params,
    )
    return deg, scat


def _dinv_col(dp):
    return lax.rsqrt(dp[:, 0:1] + dp[:, 1:2] + 1.0)


def _prep_body(x_ref, w1_ref, dp_ref, hp_ref):
    dinv = _dinv_col(dp_ref[...])
    h = jnp.dot(x_ref[...], w1_ref[...], preferred_element_type=jnp.float32)
    hp = h * dinv
    hp_ref[0] = hp[:, :HALF]
    hp_ref[1] = hp[:, HALF:]


def _bn(a, gamma, beta):
    mu = jnp.mean(a, axis=0, keepdims=True)
    var = jnp.mean((a - mu) * (a - mu), axis=0, keepdims=True)
    return (a - mu) * lax.rsqrt(var + 1e-5) * gamma + beta


def _mid_body(s_ref, dp_ref, b1_ref, g1_ref, be1_ref, w2_ref, out_ref):
    dinv = _dinv_col(dp_ref[...])
    sc = jnp.concatenate([s_ref[0], s_ref[1]], axis=1)
    a = sc * dinv + b1_ref[...]
    hr = jnp.maximum(_bn(a, g1_ref[...], be1_ref[...]), 0.0)
    h2 = jnp.dot(hr, w2_ref[...], preferred_element_type=jnp.float32)
    hp2 = h2 * dinv
    out_ref[0] = hp2[:, :HALF]
    out_ref[1] = hp2[:, HALF:]


def _final_body(s_ref, dp_ref, b2_ref, g2_ref, be2_ref, batch_ref, bb_ref,
                lw1_ref, lb1_ref, g3_ref, be3_ref, lw2_ref, lb2_ref,
                out_ref, pooled_ref, h_ref, bm_ref):
    dinv = _dinv_col(dp_ref[...])
    sc = jnp.concatenate([s_ref[0], s_ref[1]], axis=1)
    a = sc * dinv + b2_ref[...]
    h_ref[...] = jnp.maximum(_bn(a, g2_ref[...], be2_ref[...]), 0.0)

    def bmax_body(b, carry):
        bm_ref[pl.ds(b, 1), :] = jnp.max(
            h_ref[pl.ds(b * PBLK, PBLK), :], axis=0, keepdims=True)
        return carry

    lax.fori_loop(0, NPB, bmax_body, 0)

    bm = bm_ref[...]
    bfirst = bb_ref[:, 0:1]
    blast = bb_ref[:, PBLK - 1:PBLK]

    def pool_body(g, carry):
        pure = jnp.logical_and(bfirst == g, blast == g)
        pp = jnp.max(jnp.where(pure, bm, -jnp.inf), axis=0, keepdims=True)
        fb = jnp.minimum(
            jnp.sum((blast < g).astype(jnp.int32)), NPB - 1)
        lb = jnp.maximum(
            jnp.sum((bfirst <= g).astype(jnp.int32)) - 1, 0)

        def edge_max(b):
            rows = h_ref[pl.ds(b * PBLK, PBLK), :]
            ids = batch_ref[pl.ds(b * PBLK, PBLK), :]
            return jnp.max(jnp.where(ids == g, rows, -jnp.inf),
                           axis=0, keepdims=True)

        m = jnp.maximum(pp, jnp.maximum(edge_max(fb), edge_max(lb)))
        pooled_ref[pl.ds(g, 1), :] = m
        return carry

    lax.fori_loop(0, G, pool_body, 0)
    pooled = pooled_ref[...]

    p1 = jnp.dot(pooled, lw1_ref[...],
                 preferred_element_type=jnp.float32) + lb1_ref[...]
    p2 = jnp.maximum(_bn(p1, g3_ref[...], be3_ref[...]), 0.0)
    logits = jnp.dot(p2, lw2_ref[...],
                     preferred_element_type=jnp.float32) + lb2_ref[...]
    mx = jnp.max(logits, axis=1, keepdims=True)
    lse = jnp.log(jnp.sum(jnp.exp(logits - mx), axis=1, keepdims=True)) + mx
    out_ref[...] = logits - lse


def kernel(x, edge_index, batch, y, W1, b1, g1, be1, W2, b2, g2, be2,
           LW1, Lb1, g3, be3, LW2, Lb2):
    src = edge_index[0]
    dst = edge_index[1]
    deg_kernel, scatter_kernel = _sc_kernels()

    dst_deg = dst.reshape(NW, DEG_CH, 16)
    rowids = jnp.arange(DR, dtype=jnp.int32)
    degs = deg_kernel(dst_deg, rowids)
    deg_pair = degs.reshape(NC, DR * DCOL)[:, :N].T

    hp1 = _prep_call(x, W1, deg_pair)

    src_r = src.reshape(NS, NCH, CHUNK)
    src32 = jnp.concatenate([src_r, src_r + N], axis=0)
    dst16 = dst.reshape(NS, NCH, CHUNK)

    s1 = scatter_kernel(hp1.reshape(NC * N, HALF), src32, dst16)

    hp2 = _mid_call(s1.reshape(NC, N, HALF), deg_pair,
                    b1[None], g1[None], be1[None], W2)
    s2 = scatter_kernel(hp2.reshape(NC * N, HALF), src32, dst16)

    out = _final_call(s2.reshape(NC, N, HALF), deg_pair,
                      b2[None], g2[None], be2[None], batch[:, None],
                      batch.reshape(NPB, PBLK),
                      LW1, Lb1[None], g3[None], be3[None], LW2, Lb2[None])
    return (out, y.astype(jnp.int32))


_prep_call = pl.pallas_call(
    _prep_body,
    out_shape=jax.ShapeDtypeStruct((NC, N, HALF), jnp.float32),
)

_mid_call = pl.pallas_call(
    _mid_body,
    out_shape=jax.ShapeDtypeStruct((NC, N, HALF), jnp.float32),
)

_final_call = pl.pallas_call(
    _final_body,
    out_shape=jax.ShapeDtypeStruct((G, C), jnp.float32),
    scratch_shapes=[
        pltpu.VMEM((G, D2), jnp.float32),
        pltpu.VMEM((N, D2), jnp.float32),
        pltpu.VMEM((NPB, D2), jnp.float32),
    ],
)


# speedup vs baseline: 23.1433x; 1.0629x over previous
import functools

import jax
import jax.numpy as jnp
from jax import lax
from jax.experimental import pallas as pl
from jax.experimental.pallas import tpu as pltpu
from jax.experimental.pallas import tpu_sc as plsc

N = 10000
E = 320000
F_IN = 128
D1 = 256
D2 = 256
D3 = 128
C = 10
G = 64

NC = 2
NS = 16
NW = NC * NS

DR = 80
DCOL = 128
EW_DEG = E // NW
DEG_CH = EW_DEG // 16

ES = E // NS
CHUNK = 80
NCH = ES // CHUNK
NBLK = 50
NB = NCH // NBLK
RPS = N // NS

HALF = 128
QTR = 64

PBLK = 80
NPB = N // PBLK

def _deg_body(dst_hbm, rowid_hbm, out_hbm, dstv, partial, rowv, shared):
    c = lax.axis_index("c")
    s = lax.axis_index("s")
    w = c * NS + s
    zeros16 = jnp.zeros((16,), jnp.float32)

    def zrow(i, carry):
        for j in range(DCOL // 16):
            partial[i, pl.ds(j * 16, 16)] = zeros16
        return carry

    lax.fori_loop(0, DR, zrow, 0)
    rps = DR // NS
    pltpu.sync_copy(partial.at[pl.ds(0, rps)], shared.at[pl.ds(s * rps, rps)])
    pltpu.sync_copy(rowid_hbm, rowv)
    pltpu.sync_copy(dst_hbm.at[w], dstv)

    ones16 = jnp.ones((16,), jnp.float32)

    def body(i, carry):
        idx = dstv[i, :]
        r = lax.shift_right_logical(idx, 7)
        col = lax.bitwise_and(idx, 127)
        plsc.addupdate_scatter(partial, [r, col], ones16)
        return carry

    lax.fori_loop(0, DEG_CH, body, 0)
    plsc.subcore_barrier()
    pltpu.sync_copy(partial, shared.at[rowv], add=True)
    plsc.subcore_barrier()
    pltpu.sync_copy(shared.at[pl.ds(s * rps, rps)],
                    out_hbm.at[pl.ds(c * DR + s * rps, rps)])


def _scatter_body(width, hp_hbm, src_hbm, dst_hbm, out_hbm,
                  srcv, dstv, buf0, buf1, buf2, buf3,
                  sem0, sem1, sem2, sem3, acc):
    del width
    c = lax.axis_index("c")
    s = lax.axis_index("s")
    w = c * NS + s
    pltpu.sync_copy(hp_hbm.at[pl.ds(c * N + s * RPS, RPS)],
                    acc.at[pl.ds(s * RPS, RPS)])
    plsc.subcore_barrier()

    bufs = (buf0, buf1, buf2, buf3)
    sems = (sem0, sem1, sem2, sem3)

    def gather(j, q):
        return pltpu.async_copy(hp_hbm.at[srcv.at[j]], bufs[q], sems[q])

    def drain_add(j, q):
        pltpu.make_async_copy(hp_hbm.at[srcv.at[j]], bufs[q], sems[q]).wait()
        pltpu.sync_copy(bufs[q], acc.at[dstv.at[j]], add=True)

    def outer(b, carry):
        pltpu.sync_copy(src_hbm.at[w, pl.ds(b * NBLK, NBLK)], srcv)
        pltpu.sync_copy(dst_hbm.at[s, pl.ds(b * NBLK, NBLK)], dstv)
        gather(0, 0)
        gather(1, 1)

        def body(k, carry2):
            j = 4 * k
            drain_add(j, 0)
            gather(j + 2, 2)
            drain_add(j + 1, 1)
            gather(j + 3, 3)
            drain_add(j + 2, 2)
            gather(j + 4, 0)
            drain_add(j + 3, 3)
            gather(j + 5, 1)
            return carry2

        lax.fori_loop(0, NBLK // 4 - 1, body, 0)
        j = NBLK - 6
        drain_add(j, 0)
        gather(j + 2, 2)
        drain_add(j + 1, 1)
        gather(j + 3, 3)
        drain_add(j + 2, 2)
        gather(j + 4, 0)
        drain_add(j + 3, 3)
        gather(j + 5, 1)
        drain_add(j + 4, 0)
        drain_add(j + 5, 1)
        return carry

    lax.fori_loop(0, NB, outer, 0)
    plsc.subcore_barrier()
    pltpu.sync_copy(acc.at[pl.ds(s * RPS, RPS)],
                    out_hbm.at[pl.ds(c * N + s * RPS, RPS)])


@functools.lru_cache(maxsize=None)
def _sc_kernels():
    mesh = plsc.VectorSubcoreMesh(
        core_axis_name="c", subcore_axis_name="s",
        num_cores=NC, num_subcores=NS)
    params = pltpu.CompilerParams(
        use_tc_tiling_on_sc=False, needs_layout_passes=False)
    deg = pl.kernel(
        _deg_body,
        out_type=jax.ShapeDtypeStruct((NC * DR, DCOL), jnp.float32),
        mesh=mesh,
        scratch_types=[
            pltpu.VMEM((DEG_CH, 16), jnp.int32),
            pltpu.VMEM((DR, DCOL), jnp.float32),
            pltpu.VMEM((DR,), jnp.int32),
            pltpu.VMEM_SHARED((DR, DCOL), jnp.float32),
        ],
        compiler_params=params,
    )
    def make_scat(width):
        return pl.kernel(
            functools.partial(_scatter_body, width),
            out_type=jax.ShapeDtypeStruct((NC * N, width), jnp.float32),
            mesh=mesh,
            scratch_types=[
                pltpu.VMEM((NBLK, CHUNK), jnp.int32),
                pltpu.VMEM((NBLK, CHUNK), jnp.int32),
                pltpu.VMEM((CHUNK, width), jnp.float32),
                pltpu.VMEM((CHUNK, width), jnp.float32),
                pltpu.VMEM((CHUNK, width), jnp.float32),
                pltpu.VMEM((CHUNK, width), jnp.float32),
                pltpu.SemaphoreType.DMA,
                pltpu.SemaphoreType.DMA,
                pltpu.SemaphoreType.DMA,
                pltpu.SemaphoreType.DMA,
                pltpu.VMEM_SHARED((N, width), jnp.float32),
            ],
            compiler_params=params,
        )

    return deg, make_scat(QTR), make_scat(HALF)


def _dinv_col(dp):
    return lax.rsqrt(dp[:, 0:1] + dp[:, 1:2] + 1.0)


def _prep_body(x_ref, dp_ref, hp_ref):
    dinv = _dinv_col(dp_ref[...])
    xp = x_ref[...] * dinv
    hp_ref[0] = xp[:, :QTR]
    hp_ref[1] = xp[:, QTR:]


def _bn(a, gamma, beta):
    mu = jnp.mean(a, axis=0, keepdims=True)
    var = jnp.mean((a - mu) * (a - mu), axis=0, keepdims=True)
    return (a - mu) * lax.rsqrt(var + 1e-5) * gamma + beta


def _mid_body(s_ref, dp_ref, w1_ref, b1_ref, g1_ref, be1_ref, w2_ref,
              out_ref):
    dinv = _dinv_col(dp_ref[...])
    t = jnp.concatenate([s_ref[0], s_ref[1]], axis=1) * dinv
    a = jnp.dot(t, w1_ref[...], precision=lax.Precision.HIGHEST,
                preferred_element_type=jnp.float32) + b1_ref[...]
    hr = jnp.maximum(_bn(a, g1_ref[...], be1_ref[...]), 0.0)
    h2 = jnp.dot(hr, w2_ref[...], preferred_element_type=jnp.float32)
    hp2 = h2 * dinv
    out_ref[0] = hp2[:, :HALF]
    out_ref[1] = hp2[:, HALF:]


def _final_body(s_ref, dp_ref, b2_ref, g2_ref, be2_ref, batch_ref, bb_ref,
                lw1_ref, lb1_ref, g3_ref, be3_ref, lw2_ref, lb2_ref,
                out_ref, pooled_ref, h_ref, bm_ref):
    dinv = _dinv_col(dp_ref[...])
    sc = jnp.concatenate([s_ref[0], s_ref[1]], axis=1)
    a = sc * dinv + b2_ref[...]
    h_ref[...] = jnp.maximum(_bn(a, g2_ref[...], be2_ref[...]), 0.0)

    def bmax_body(b, carry):
        bm_ref[pl.ds(b, 1), :] = jnp.max(
            h_ref[pl.ds(b * PBLK, PBLK), :], axis=0, keepdims=True)
        return carry

    lax.fori_loop(0, NPB, bmax_body, 0)

    bm = bm_ref[...]
    bfirst = bb_ref[:, 0:1]
    blast = bb_ref[:, PBLK - 1:PBLK]

    def pool_body(g, carry):
        pure = jnp.logical_and(bfirst == g, blast == g)
        pp = jnp.max(jnp.where(pure, bm, -jnp.inf), axis=0, keepdims=True)
        fb = jnp.minimum(
            jnp.sum((blast < g).astype(jnp.int32)), NPB - 1)
        lb = jnp.maximum(
            jnp.sum((bfirst <= g).astype(jnp.int32)) - 1, 0)

        def edge_max(b):
            rows = h_ref[pl.ds(b * PBLK, PBLK), :]
            ids = batch_ref[pl.ds(b * PBLK, PBLK), :]
            return jnp.max(jnp.where(ids == g, rows, -jnp.inf),
                           axis=0, keepdims=True)

        m = jnp.maximum(pp, jnp.maximum(edge_max(fb), edge_max(lb)))
        pooled_ref[pl.ds(g, 1), :] = m
        return carry

    lax.fori_loop(0, G, pool_body, 0)
    pooled = pooled_ref[...]

    p1 = jnp.dot(pooled, lw1_ref[...],
                 preferred_element_type=jnp.float32) + lb1_ref[...]
    p2 = jnp.maximum(_bn(p1, g3_ref[...], be3_ref[...]), 0.0)
    logits = jnp.dot(p2, lw2_ref[...],
                     preferred_element_type=jnp.float32) + lb2_ref[...]
    mx = jnp.max(logits, axis=1, keepdims=True)
    lse = jnp.log(jnp.sum(jnp.exp(logits - mx), axis=1, keepdims=True)) + mx
    out_ref[...] = logits - lse


def kernel(x, edge_index, batch, y, W1, b1, g1, be1, W2, b2, g2, be2,
           LW1, Lb1, g3, be3, LW2, Lb2):
    src = edge_index[0]
    dst = edge_index[1]
    deg_kernel, scat64, scat128 = _sc_kernels()

    dst_deg = dst.reshape(NW, DEG_CH, 16)
    rowids = jnp.arange(DR, dtype=jnp.int32)
    degs = deg_kernel(dst_deg, rowids)
    deg_pair = degs.reshape(NC, DR * DCOL)[:, :N].T

    xp = _prep_call(x, deg_pair)

    src_r = src.reshape(NS, NCH, CHUNK)
    src32 = jnp.concatenate([src_r, src_r + N], axis=0)
    dst16 = dst.reshape(NS, NCH, CHUNK)

    s1 = scat64(xp.reshape(NC * N, QTR), src32, dst16)

    hp2 = _mid_call(s1.reshape(NC, N, QTR), deg_pair,
                    W1, b1[None], g1[None], be1[None], W2)
    s2 = scat128(hp2.reshape(NC * N, HALF), src32, dst16)

    out = _final_call(s2.reshape(NC, N, HALF), deg_pair,
                      b2[None], g2[None], be2[None], batch[:, None],
                      batch.reshape(NPB, PBLK),
                      LW1, Lb1[None], g3[None], be3[None], LW2, Lb2[None])
    return (out, y.astype(jnp.int32))


_prep_call = pl.pallas_call(
    _prep_body,
    out_shape=jax.ShapeDtypeStruct((NC, N, QTR), jnp.float32),
)

_mid_call = pl.pallas_call(
    _mid_body,
    out_shape=jax.ShapeDtypeStruct((NC, N, HALF), jnp.float32),
)

_final_call = pl.pallas_call(
    _final_body,
    out_shape=jax.ShapeDtypeStruct((G, C), jnp.float32),
    scratch_shapes=[
        pltpu.VMEM((G, D2), jnp.float32),
        pltpu.VMEM((N, D2), jnp.float32),
        pltpu.VMEM((NPB, D2), jnp.float32),
    ],
)


# speedup vs baseline: 23.9618x; 1.0354x over previous
import functools

import jax
import jax.numpy as jnp
from jax import lax
from jax.experimental import pallas as pl
from jax.experimental.pallas import tpu as pltpu
from jax.experimental.pallas import tpu_sc as plsc

N = 10000
E = 320000
F_IN = 128
D1 = 256
D2 = 256
D3 = 128
C = 10
G = 64

NC = 2
NS = 16
NW = NC * NS

DR = 80
DCOL = 128
EW_DEG = E // NW
DEG_CH = EW_DEG // 16

ES = E // NS
CHUNK = 80
NCH = ES // CHUNK
NBLK = 50
NB = NCH // NBLK
CHUNK_Q = 100
NCH_Q = ES // CHUNK_Q
NBLK_Q = 50
NB_Q = NCH_Q // NBLK_Q
RPS = N // NS

HALF = 128
QTR = 64

PBLK = 80
NPB = N // PBLK

def _deg_body(dst_hbm, rowid_hbm, out_hbm, dstv, partial, rowv, shared):
    c = lax.axis_index("c")
    s = lax.axis_index("s")
    w = c * NS + s
    zeros16 = jnp.zeros((16,), jnp.float32)

    def zrow(i, carry):
        for j in range(DCOL // 16):
            partial[i, pl.ds(j * 16, 16)] = zeros16
        return carry

    lax.fori_loop(0, DR, zrow, 0)
    rps = DR // NS
    pltpu.sync_copy(partial.at[pl.ds(0, rps)], shared.at[pl.ds(s * rps, rps)])
    pltpu.sync_copy(rowid_hbm, rowv)
    pltpu.sync_copy(dst_hbm.at[w], dstv)

    ones16 = jnp.ones((16,), jnp.float32)

    def body(i, carry):
        idx = dstv[i, :]
        r = lax.shift_right_logical(idx, 7)
        col = lax.bitwise_and(idx, 127)
        plsc.addupdate_scatter(partial, [r, col], ones16)
        return carry

    lax.fori_loop(0, DEG_CH, body, 0)
    plsc.subcore_barrier()
    pltpu.sync_copy(partial, shared.at[rowv], add=True)
    plsc.subcore_barrier()
    pltpu.sync_copy(shared.at[pl.ds(s * rps, rps)],
                    out_hbm.at[pl.ds(c * DR + s * rps, rps)])


def _scatter_body(nblk, nb, hp_hbm, src_hbm, dst_hbm, out_hbm,
                  srcv, dstv, buf0, buf1, buf2, buf3,
                  sem0, sem1, sem2, sem3, acc):
    c = lax.axis_index("c")
    s = lax.axis_index("s")
    w = c * NS + s
    pltpu.sync_copy(hp_hbm.at[pl.ds(c * N + s * RPS, RPS)],
                    acc.at[pl.ds(s * RPS, RPS)])
    plsc.subcore_barrier()

    bufs = (buf0, buf1, buf2, buf3)
    sems = (sem0, sem1, sem2, sem3)

    def gather(j, q):
        return pltpu.async_copy(hp_hbm.at[srcv.at[j]], bufs[q], sems[q])

    def drain_add(j, q):
        pltpu.make_async_copy(hp_hbm.at[srcv.at[j]], bufs[q], sems[q]).wait()
        pltpu.sync_copy(bufs[q], acc.at[dstv.at[j]], add=True)

    def outer(b, carry):
        pltpu.sync_copy(src_hbm.at[w, pl.ds(b * nblk, nblk)], srcv)
        pltpu.sync_copy(dst_hbm.at[s, pl.ds(b * nblk, nblk)], dstv)
        gather(0, 0)
        gather(1, 1)

        def body(k, carry2):
            j = 4 * k
            drain_add(j, 0)
            gather(j + 2, 2)
            drain_add(j + 1, 1)
            gather(j + 3, 3)
            drain_add(j + 2, 2)
            gather(j + 4, 0)
            drain_add(j + 3, 3)
            gather(j + 5, 1)
            return carry2

        lax.fori_loop(0, nblk // 4 - 1, body, 0)
        j = nblk - 6
        drain_add(j, 0)
        gather(j + 2, 2)
        drain_add(j + 1, 1)
        gather(j + 3, 3)
        drain_add(j + 2, 2)
        gather(j + 4, 0)
        drain_add(j + 3, 3)
        gather(j + 5, 1)
        drain_add(j + 4, 0)
        drain_add(j + 5, 1)
        return carry

    lax.fori_loop(0, nb, outer, 0)
    plsc.subcore_barrier()
    pltpu.sync_copy(acc.at[pl.ds(s * RPS, RPS)],
                    out_hbm.at[pl.ds(c * N + s * RPS, RPS)])


@functools.lru_cache(maxsize=None)
def _sc_kernels():
    mesh = plsc.VectorSubcoreMesh(
        core_axis_name="c", subcore_axis_name="s",
        num_cores=NC, num_subcores=NS)
    params = pltpu.CompilerParams(
        use_tc_tiling_on_sc=False, needs_layout_passes=False)
    deg = pl.kernel(
        _deg_body,
        out_type=jax.ShapeDtypeStruct((NC * DR, DCOL), jnp.float32),
        mesh=mesh,
        scratch_types=[
            pltpu.VMEM((DEG_CH, 16), jnp.int32),
            pltpu.VMEM((DR, DCOL), jnp.float32),
            pltpu.VMEM((DR,), jnp.int32),
            pltpu.VMEM_SHARED((DR, DCOL), jnp.float32),
        ],
        compiler_params=params,
    )
    def make_scat(width, chunk, nblk, nb):
        return pl.kernel(
            functools.partial(_scatter_body, nblk, nb),
            out_type=jax.ShapeDtypeStruct((NC * N, width), jnp.float32),
            mesh=mesh,
            scratch_types=[
                pltpu.VMEM((nblk, chunk), jnp.int32),
                pltpu.VMEM((nblk, chunk), jnp.int32),
                pltpu.VMEM((chunk, width), jnp.float32),
                pltpu.VMEM((chunk, width), jnp.float32),
                pltpu.VMEM((chunk, width), jnp.float32),
                pltpu.VMEM((chunk, width), jnp.float32),
                pltpu.SemaphoreType.DMA,
                pltpu.SemaphoreType.DMA,
                pltpu.SemaphoreType.DMA,
                pltpu.SemaphoreType.DMA,
                pltpu.VMEM_SHARED((N, width), jnp.float32),
            ],
            compiler_params=params,
        )

    return (deg, make_scat(QTR, CHUNK_Q, NBLK_Q, NB_Q),
            make_scat(HALF, CHUNK, NBLK, NB))


def _dinv_col(dp):
    return lax.rsqrt(dp[:, 0:1] + dp[:, 1:2] + 1.0)


def _prep_body(x_ref, dp_ref, hp_ref):
    dinv = _dinv_col(dp_ref[...])
    xp = x_ref[...] * dinv
    hp_ref[0] = xp[:, :QTR]
    hp_ref[1] = xp[:, QTR:]


def _bn(a, gamma, beta):
    mu = jnp.mean(a, axis=0, keepdims=True)
    var = jnp.mean((a - mu) * (a - mu), axis=0, keepdims=True)
    return (a - mu) * lax.rsqrt(var + 1e-5) * gamma + beta


def _mid_body(s_ref, dp_ref, w1_ref, b1_ref, g1_ref, be1_ref, w2_ref,
              out_ref):
    dinv = _dinv_col(dp_ref[...])
    t = jnp.concatenate([s_ref[0], s_ref[1]], axis=1) * dinv
    a = jnp.dot(t, w1_ref[...], precision=lax.Precision.HIGHEST,
                preferred_element_type=jnp.float32) + b1_ref[...]
    hr = jnp.maximum(_bn(a, g1_ref[...], be1_ref[...]), 0.0)
    h2 = jnp.dot(hr, w2_ref[...], preferred_element_type=jnp.float32)
    hp2 = h2 * dinv
    out_ref[0] = hp2[:, :HALF]
    out_ref[1] = hp2[:, HALF:]


def _final_body(s_ref, dp_ref, b2_ref, g2_ref, be2_ref, batch_ref, bb_ref,
                lw1_ref, lb1_ref, g3_ref, be3_ref, lw2_ref, lb2_ref,
                out_ref, pooled_ref, h_ref, bm_ref):
    dinv = _dinv_col(dp_ref[...])
    sc = jnp.concatenate([s_ref[0], s_ref[1]], axis=1)
    a = sc * dinv + b2_ref[...]
    h_ref[...] = jnp.maximum(_bn(a, g2_ref[...], be2_ref[...]), 0.0)

    def bmax_body(b, carry):
        bm_ref[pl.ds(b, 1), :] = jnp.max(
            h_ref[pl.ds(b * PBLK, PBLK), :], axis=0, keepdims=True)
        return carry

    lax.fori_loop(0, NPB, bmax_body, 0)

    bm = bm_ref[...]
    bfirst = bb_ref[:, 0:1]
    blast = bb_ref[:, PBLK - 1:PBLK]

    def pool_body(g, carry):
        pure = jnp.logical_and(bfirst == g, blast == g)
        pp = jnp.max(jnp.where(pure, bm, -jnp.inf), axis=0, keepdims=True)
        fb = jnp.minimum(
            jnp.sum((blast < g).astype(jnp.int32)), NPB - 1)
        lb = jnp.maximum(
            jnp.sum((bfirst <= g).astype(jnp.int32)) - 1, 0)

        def edge_max(b):
            rows = h_ref[pl.ds(b * PBLK, PBLK), :]
            ids = batch_ref[pl.ds(b * PBLK, PBLK), :]
            return jnp.max(jnp.where(ids == g, rows, -jnp.inf),
                           axis=0, keepdims=True)

        m = jnp.maximum(pp, jnp.maximum(edge_max(fb), edge_max(lb)))
        pooled_ref[pl.ds(g, 1), :] = m
        return carry

    lax.fori_loop(0, G, pool_body, 0)
    pooled = pooled_ref[...]

    p1 = jnp.dot(pooled, lw1_ref[...],
                 preferred_element_type=jnp.float32) + lb1_ref[...]
    p2 = jnp.maximum(_bn(p1, g3_ref[...], be3_ref[...]), 0.0)
    logits = jnp.dot(p2, lw2_ref[...],
                     preferred_element_type=jnp.float32) + lb2_ref[...]
    mx = jnp.max(logits, axis=1, keepdims=True)
    lse = jnp.log(jnp.sum(jnp.exp(logits - mx), axis=1, keepdims=True)) + mx
    out_ref[...] = logits - lse


def kernel(x, edge_index, batch, y, W1, b1, g1, be1, W2, b2, g2, be2,
           LW1, Lb1, g3, be3, LW2, Lb2):
    src = edge_index[0]
    dst = edge_index[1]
    deg_kernel, scat64, scat128 = _sc_kernels()

    dst_deg = dst.reshape(NW, DEG_CH, 16)
    rowids = jnp.arange(DR, dtype=jnp.int32)
    degs = deg_kernel(dst_deg, rowids)
    deg_pair = degs.reshape(NC, DR * DCOL)[:, :N].T

    xp = _prep_call(x, deg_pair)

    src_q = src.reshape(NS, NCH_Q, CHUNK_Q)
    src32_q = jnp.concatenate([src_q, src_q + N], axis=0)
    dst16_q = dst.reshape(NS, NCH_Q, CHUNK_Q)
    s1 = scat64(xp.reshape(NC * N, QTR), src32_q, dst16_q)

    hp2 = _mid_call(s1.reshape(NC, N, QTR), deg_pair,
                    W1, b1[None], g1[None], be1[None], W2)
    src_r = src.reshape(NS, NCH, CHUNK)
    src32 = jnp.concatenate([src_r, src_r + N], axis=0)
    dst16 = dst.reshape(NS, NCH, CHUNK)
    s2 = scat128(hp2.reshape(NC * N, HALF), src32, dst16)

    out = _final_call(s2.reshape(NC, N, HALF), deg_pair,
                      b2[None], g2[None], be2[None], batch[:, None],
                      batch.reshape(NPB, PBLK),
                      LW1, Lb1[None], g3[None], be3[None], LW2, Lb2[None])
    return (out, y.astype(jnp.int32))


_prep_call = pl.pallas_call(
    _prep_body,
    out_shape=jax.ShapeDtypeStruct((NC, N, QTR), jnp.float32),
)

_mid_call = pl.pallas_call(
    _mid_body,
    out_shape=jax.ShapeDtypeStruct((NC, N, HALF), jnp.float32),
)

_final_call = pl.pallas_call(
    _final_body,
    out_shape=jax.ShapeDtypeStruct((G, C), jnp.float32),
    scratch_shapes=[
        pltpu.VMEM((G, D2), jnp.float32),
        pltpu.VMEM((N, D2), jnp.float32),
        pltpu.VMEM((NPB, D2), jnp.float32),
    ],
)
